# Initial kernel scaffold; baseline (speedup 1.0000x reference)
import jax
import jax.numpy as jnp
from jax.experimental import pallas as pl


def kernel(x, edge_index, W1, a_src1, a_dst1, b1, W2, a_src2, a_dst2, b2, W3, a_src3, a_dst3, b3):
    raise NotImplementedError("write your pallas kernel here")


# speedup vs baseline: 4.1040x; 4.1040x over previous
import functools
import math

import jax
import jax.numpy as jnp
from jax.experimental import pallas as pl


def _amat(a, h_true, hp):
    h, c = a.shape[1], a.shape[2]
    eye = jnp.eye(h, dtype=a.dtype)
    m = jnp.einsum('hc,hg->hcg', a[0], eye).reshape(h * c, h)
    if h < hp:
        m = jnp.tile(m, (1, hp // h))
    return m


def _mm_first_kernel(x_ref, w_ref, asrc_ref, adst_ref, out_ref, *, d, hp):
    h = jnp.dot(x_ref[...], w_ref[...], preferred_element_type=jnp.float32)
    out_ref[:, :d] = h
    out_ref[:, d:d + hp] = jnp.dot(h, asrc_ref[...],
                                   preferred_element_type=jnp.float32)
    out_ref[:, d + hp:] = jnp.dot(h, adst_ref[...],
                                  preferred_element_type=jnp.float32)


def _mm_next_kernel(num_ref, den_ref, b_ref, w_ref, asrc_ref, adst_ref,
                    out_ref, *, hin, cin, d, hp):
    b = b_ref[...]
    parts = [
        jnp.maximum(
            num_ref[:, h0 * cin:(h0 + 1) * cin]
            / (den_ref[:, h0:h0 + 1] + 1e-16)
            + b[:, h0 * cin:(h0 + 1) * cin], 0.0)
        for h0 in range(hin)
    ]
    xin = jnp.concatenate(parts, axis=1)
    h = jnp.dot(xin, w_ref[...], preferred_element_type=jnp.float32)
    out_ref[:, :d] = h
    out_ref[:, d:d + hp] = jnp.dot(h, asrc_ref[...],
                                   preferred_element_type=jnp.float32)
    out_ref[:, d + hp:] = jnp.dot(h, adst_ref[...],
                                  preferred_element_type=jnp.float32)


def _edge_kernel(sidx_ref, didx_ref, ha_ref, num_ref, den_ref,
                 *, n, d, ht, c, hp, eb, ck):
    i = pl.program_id(0)

    @pl.when(i == 0)
    def _():
        num_ref[...] = jnp.zeros_like(num_ref)
        den_ref[...] = jnp.zeros_like(den_ref)

    s_b = sidx_ref[0]
    d_b = didx_ref[0]
    ew = eb + 8
    w0 = jnp.maximum(jnp.minimum((jnp.min(d_b) // 8) * 8, n - ew), 0)
    w0 = pl.multiple_of(w0, 8)

    wio = jax.lax.broadcasted_iota(jnp.int32, (ew, eb), 0) + w0
    ohd_t = (jnp.broadcast_to(d_b, (ew, eb)) == wio).astype(jnp.float32)

    ad_w = ha_ref[pl.ds(w0, ew), d + hp:]
    ad_g = jax.lax.dot_general(ohd_t, ad_w, (((0,), (0,)), ((), ())),
                               preferred_element_type=jnp.float32)

    def body(k, acc):
        cio = jax.lax.broadcasted_iota(jnp.int32, (ck, eb), 0) + k * ck
        ohs_t = (jnp.broadcast_to(s_b, (ck, eb)) == cio).astype(jnp.float32)
        blk = ha_ref[pl.ds(pl.multiple_of(k * ck, 8), ck), :d + hp]
        return acc + jax.lax.dot_general(
            ohs_t, blk, (((0,), (0,)), ((), ())),
            preferred_element_type=jnp.float32)

    g = jax.lax.fori_loop(0, n // ck, body,
                          jnp.zeros((eb, d + hp), jnp.float32))
    h_g = g[:, :d]
    as_g = g[:, d:]

    e = as_g + ad_g
    e = jnp.where(e >= 0, e, 0.2 * e)
    ex = jnp.exp(e)

    parts = [h_g[:, h0 * c:(h0 + 1) * c] * ex[:, h0:h0 + 1]
             for h0 in range(ht)]
    weighted = parts[0] if ht == 1 else jnp.concatenate(parts, axis=1)

    num_ref[pl.ds(w0, ew), :] += jax.lax.dot_general(
        ohd_t, weighted, (((1,), (0,)), ((), ())),
        preferred_element_type=jnp.float32)
    den_ref[pl.ds(w0, ew), :] += jax.lax.dot_general(
        ohd_t, ex, (((1,), (0,)), ((), ())),
        preferred_element_type=jnp.float32)


def _final_kernel(num_ref, den_ref, b_ref, out_ref):
    l = num_ref[...] / (den_ref[:, 0:1] + 1e-16) + b_ref[...]
    m = jnp.max(l, axis=1, keepdims=True)
    lse = jnp.log(jnp.sum(jnp.exp(l - m), axis=1, keepdims=True)) + m
    out_ref[...] = l - lse


def _run_dense_first(x, w, asrc, adst, bn, d, hp):
    n, fin = x.shape
    grid = n // bn
    return pl.pallas_call(
        functools.partial(_mm_first_kernel, d=d, hp=hp),
        grid=(grid,),
        in_specs=[
            pl.BlockSpec((bn, fin), lambda i: (i, 0)),
            pl.BlockSpec((fin, d), lambda i: (0, 0)),
            pl.BlockSpec((d, hp), lambda i: (0, 0)),
            pl.BlockSpec((d, hp), lambda i: (0, 0)),
        ],
        out_specs=pl.BlockSpec((bn, d + 2 * hp), lambda i: (i, 0)),
        out_shape=jax.ShapeDtypeStruct((n, d + 2 * hp), jnp.float32),
    )(x, w, asrc, adst)


def _run_dense_next(num, den, b, w, asrc, adst, bn, hin, cin, d, hp):
    n, din = num.shape
    grid = n // bn
    return pl.pallas_call(
        functools.partial(_mm_next_kernel, hin=hin, cin=cin, d=d, hp=hp),
        grid=(grid,),
        in_specs=[
            pl.BlockSpec((bn, din), lambda i: (i, 0)),
            pl.BlockSpec((bn, hp), lambda i: (i, 0)),
            pl.BlockSpec((1, din), lambda i: (0, 0)),
            pl.BlockSpec((din, d), lambda i: (0, 0)),
            pl.BlockSpec((d, hp), lambda i: (0, 0)),
            pl.BlockSpec((d, hp), lambda i: (0, 0)),
        ],
        out_specs=pl.BlockSpec((bn, d + 2 * hp), lambda i: (i, 0)),
        out_shape=jax.ShapeDtypeStruct((n, d + 2 * hp), jnp.float32),
    )(num, den, b, w, asrc, adst)


def _run_edges(sidx, didx, ha, n, d, ht, c, hp, eb, ck):
    nblk = sidx.shape[0]
    return pl.pallas_call(
        functools.partial(_edge_kernel, n=n, d=d, ht=ht, c=c, hp=hp,
                          eb=eb, ck=ck),
        grid=(nblk,),
        in_specs=[
            pl.BlockSpec((1, 1, eb), lambda i: (i, 0, 0)),
            pl.BlockSpec((1, 1, eb), lambda i: (i, 0, 0)),
            pl.BlockSpec((n, d + 2 * hp), lambda i: (0, 0)),
        ],
        out_specs=[
            pl.BlockSpec((n, d), lambda i: (0, 0)),
            pl.BlockSpec((n, hp), lambda i: (0, 0)),
        ],
        out_shape=[
            jax.ShapeDtypeStruct((n, d), jnp.float32),
            jax.ShapeDtypeStruct((n, hp), jnp.float32),
        ],
    )(sidx, didx, ha)


def _run_final(num, den, b, bn):
    n, d = num.shape
    hp = den.shape[1]
    grid = n // bn
    return pl.pallas_call(
        _final_kernel,
        grid=(grid,),
        in_specs=[
            pl.BlockSpec((bn, d), lambda i: (i, 0)),
            pl.BlockSpec((bn, hp), lambda i: (i, 0)),
            pl.BlockSpec((1, d), lambda i: (0, 0)),
        ],
        out_specs=pl.BlockSpec((bn, d), lambda i: (i, 0)),
        out_shape=jax.ShapeDtypeStruct((n, d), jnp.float32),
    )(num, den, b)


def kernel(x, edge_index, W1, a_src1, a_dst1, b1, W2, a_src2, a_dst2, b2,
           W3, a_src3, a_dst3, b3):
    n = x.shape[0]
    e = edge_index.shape[1]
    hp = 8
    heads = a_src1.shape[1]
    hid = a_src1.shape[2]
    out_d = a_src3.shape[2]
    d12 = heads * hid

    loop = jnp.arange(n, dtype=edge_index.dtype)
    src = jnp.concatenate([edge_index[0], loop])
    dst = jnp.concatenate([edge_index[1], loop])
    order = jnp.argsort(dst)
    src = src[order]
    dst = dst[order]

    eb = 512 if n >= 1024 else n
    total = e + n
    nblk = math.ceil(total / eb)
    pad = nblk * eb - total
    src = jnp.concatenate([src, jnp.zeros((pad,), src.dtype)])
    dst = jnp.concatenate([dst, jnp.full((pad,), n, dst.dtype)])
    sidx = src.reshape(nblk, 1, eb)
    didx = dst.reshape(nblk, 1, eb)

    bn = 1000 if n % 1000 == 0 else n
    ck = 1000 if n % 1000 == 0 else n

    a1s, a1d = _amat(a_src1, heads, hp), _amat(a_dst1, heads, hp)
    a2s, a2d = _amat(a_src2, heads, hp), _amat(a_dst2, heads, hp)
    a3s, a3d = _amat(a_src3, 1, hp), _amat(a_dst3, 1, hp)

    ha1 = _run_dense_first(x, W1, a1s, a1d, bn, d12, hp)
    num1, den1 = _run_edges(sidx, didx, ha1, n, d12, heads, hid, hp, eb, ck)

    ha2 = _run_dense_next(num1, den1, b1.reshape(1, -1), W2, a2s, a2d,
                          bn, heads, hid, d12, hp)
    num2, den2 = _run_edges(sidx, didx, ha2, n, d12, heads, hid, hp, eb, ck)

    ha3 = _run_dense_next(num2, den2, b2.reshape(1, -1), W3, a3s, a3d,
                          bn, heads, hid, out_d, hp)
    num3, den3 = _run_edges(sidx, didx, ha3, n, out_d, 1, out_d, hp, eb, ck)

    return _run_final(num3, den3, b3.reshape(1, -1), bn)


# speedup vs baseline: 4.2369x; 1.0324x over previous
import functools
import math

import jax
import jax.numpy as jnp
from jax.experimental import pallas as pl


def _amat(a, h_true, hp):
    h, c = a.shape[1], a.shape[2]
    eye = jnp.eye(h, dtype=a.dtype)
    m = jnp.einsum('hc,hg->hcg', a[0], eye).reshape(h * c, h)
    if h < hp:
        m = jnp.tile(m, (1, hp // h))
    return m


def _mm_first_kernel(x_ref, w_ref, asrc_ref, adst_ref, out_ref, *, d, hp):
    h = jnp.dot(x_ref[...], w_ref[...], preferred_element_type=jnp.float32)
    out_ref[:, :d] = h
    out_ref[:, d:d + hp] = jnp.dot(h, asrc_ref[...],
                                   preferred_element_type=jnp.float32)
    out_ref[:, d + hp:] = jnp.dot(h, adst_ref[...],
                                  preferred_element_type=jnp.float32)


def _mm_next_kernel(num_ref, den_ref, b_ref, w_ref, asrc_ref, adst_ref,
                    out_ref, *, hin, cin, d, hp):
    b = b_ref[...]
    parts = [
        jnp.maximum(
            num_ref[:, h0 * cin:(h0 + 1) * cin]
            / (den_ref[:, h0:h0 + 1] + 1e-16)
            + b[:, h0 * cin:(h0 + 1) * cin], 0.0)
        for h0 in range(hin)
    ]
    xin = jnp.concatenate(parts, axis=1)
    h = jnp.dot(xin, w_ref[...], preferred_element_type=jnp.float32)
    out_ref[:, :d] = h
    out_ref[:, d:d + hp] = jnp.dot(h, asrc_ref[...],
                                   preferred_element_type=jnp.float32)
    out_ref[:, d + hp:] = jnp.dot(h, adst_ref[...],
                                  preferred_element_type=jnp.float32)


def _edge_kernel(sidx_ref, didx_ref, ha_ref, num_ref, den_ref,
                 *, n, d, ht, c, hp, eb, ck):
    i = pl.program_id(0)

    @pl.when(i == 0)
    def _():
        num_ref[...] = jnp.zeros_like(num_ref)
        den_ref[...] = jnp.zeros_like(den_ref)

    s_b = sidx_ref[0]
    d_b = didx_ref[0]
    ew = eb + 8
    w0 = jnp.maximum(jnp.minimum((jnp.min(d_b) // 8) * 8, n - ew), 0)
    w0 = pl.multiple_of(w0, 8)

    wio = jax.lax.broadcasted_iota(jnp.int32, (ew, eb), 0) + w0
    ohd_t = (jnp.broadcast_to(d_b, (ew, eb)) == wio).astype(jnp.float32)

    ad_w = ha_ref[pl.ds(w0, ew), d + hp:]
    ad_g = jax.lax.dot_general(ohd_t, ad_w, (((0,), (0,)), ((), ())),
                               preferred_element_type=jnp.float32)

    def body(k, acc):
        cio = jax.lax.broadcasted_iota(jnp.int32, (ck, eb), 0) + k * ck
        ohs_t = (jnp.broadcast_to(s_b, (ck, eb)) == cio).astype(jnp.bfloat16)
        blk = ha_ref[pl.ds(pl.multiple_of(k * ck, 8), ck), :d + hp]
        return acc + jax.lax.dot_general(
            ohs_t, blk.astype(jnp.bfloat16), (((0,), (0,)), ((), ())),
            preferred_element_type=jnp.float32)

    g = jax.lax.fori_loop(0, n // ck, body,
                          jnp.zeros((eb, d + hp), jnp.float32))
    h_g = g[:, :d]
    as_g = g[:, d:]

    e = as_g + ad_g
    e = jnp.where(e >= 0, e, 0.2 * e)
    ex = jnp.exp(e)

    parts = [h_g[:, h0 * c:(h0 + 1) * c] * ex[:, h0:h0 + 1]
             for h0 in range(ht)]
    weighted = parts[0] if ht == 1 else jnp.concatenate(parts, axis=1)

    num_ref[pl.ds(w0, ew), :] += jax.lax.dot_general(
        ohd_t, weighted, (((1,), (0,)), ((), ())),
        preferred_element_type=jnp.float32)
    den_ref[pl.ds(w0, ew), :] += jax.lax.dot_general(
        ohd_t, ex, (((1,), (0,)), ((), ())),
        preferred_element_type=jnp.float32)


def _final_kernel(num_ref, den_ref, b_ref, out_ref):
    l = num_ref[...] / (den_ref[:, 0:1] + 1e-16) + b_ref[...]
    m = jnp.max(l, axis=1, keepdims=True)
    lse = jnp.log(jnp.sum(jnp.exp(l - m), axis=1, keepdims=True)) + m
    out_ref[...] = l - lse


def _run_dense_first(x, w, asrc, adst, bn, d, hp):
    n, fin = x.shape
    grid = n // bn
    return pl.pallas_call(
        functools.partial(_mm_first_kernel, d=d, hp=hp),
        grid=(grid,),
        in_specs=[
            pl.BlockSpec((bn, fin), lambda i: (i, 0)),
            pl.BlockSpec((fin, d), lambda i: (0, 0)),
            pl.BlockSpec((d, hp), lambda i: (0, 0)),
            pl.BlockSpec((d, hp), lambda i: (0, 0)),
        ],
        out_specs=pl.BlockSpec((bn, d + 2 * hp), lambda i: (i, 0)),
        out_shape=jax.ShapeDtypeStruct((n, d + 2 * hp), jnp.float32),
    )(x, w, asrc, adst)


def _run_dense_next(num, den, b, w, asrc, adst, bn, hin, cin, d, hp):
    n, din = num.shape
    grid = n // bn
    return pl.pallas_call(
        functools.partial(_mm_next_kernel, hin=hin, cin=cin, d=d, hp=hp),
        grid=(grid,),
        in_specs=[
            pl.BlockSpec((bn, din), lambda i: (i, 0)),
            pl.BlockSpec((bn, hp), lambda i: (i, 0)),
            pl.BlockSpec((1, din), lambda i: (0, 0)),
            pl.BlockSpec((din, d), lambda i: (0, 0)),
            pl.BlockSpec((d, hp), lambda i: (0, 0)),
            pl.BlockSpec((d, hp), lambda i: (0, 0)),
        ],
        out_specs=pl.BlockSpec((bn, d + 2 * hp), lambda i: (i, 0)),
        out_shape=jax.ShapeDtypeStruct((n, d + 2 * hp), jnp.float32),
    )(num, den, b, w, asrc, adst)


def _run_edges(sidx, didx, ha, n, d, ht, c, hp, eb, ck):
    nblk = sidx.shape[0]
    return pl.pallas_call(
        functools.partial(_edge_kernel, n=n, d=d, ht=ht, c=c, hp=hp,
                          eb=eb, ck=ck),
        grid=(nblk,),
        in_specs=[
            pl.BlockSpec((1, 1, eb), lambda i: (i, 0, 0)),
            pl.BlockSpec((1, 1, eb), lambda i: (i, 0, 0)),
            pl.BlockSpec((n, d + 2 * hp), lambda i: (0, 0)),
        ],
        out_specs=[
            pl.BlockSpec((n, d), lambda i: (0, 0)),
            pl.BlockSpec((n, hp), lambda i: (0, 0)),
        ],
        out_shape=[
            jax.ShapeDtypeStruct((n, d), jnp.float32),
            jax.ShapeDtypeStruct((n, hp), jnp.float32),
        ],
    )(sidx, didx, ha)


def _run_final(num, den, b, bn):
    n, d = num.shape
    hp = den.shape[1]
    grid = n // bn
    return pl.pallas_call(
        _final_kernel,
        grid=(grid,),
        in_specs=[
            pl.BlockSpec((bn, d), lambda i: (i, 0)),
            pl.BlockSpec((bn, hp), lambda i: (i, 0)),
            pl.BlockSpec((1, d), lambda i: (0, 0)),
        ],
        out_specs=pl.BlockSpec((bn, d), lambda i: (i, 0)),
        out_shape=jax.ShapeDtypeStruct((n, d), jnp.float32),
    )(num, den, b)


def kernel(x, edge_index, W1, a_src1, a_dst1, b1, W2, a_src2, a_dst2, b2,
           W3, a_src3, a_dst3, b3):
    n = x.shape[0]
    e = edge_index.shape[1]
    hp = 8
    heads = a_src1.shape[1]
    hid = a_src1.shape[2]
    out_d = a_src3.shape[2]
    d12 = heads * hid

    loop = jnp.arange(n, dtype=edge_index.dtype)
    src = jnp.concatenate([edge_index[0], loop])
    dst = jnp.concatenate([edge_index[1], loop])
    order = jnp.argsort(dst)
    src = src[order]
    dst = dst[order]

    eb = 512 if n >= 1024 else n
    total = e + n
    nblk = math.ceil(total / eb)
    pad = nblk * eb - total
    src = jnp.concatenate([src, jnp.zeros((pad,), src.dtype)])
    dst = jnp.concatenate([dst, jnp.full((pad,), n, dst.dtype)])
    sidx = src.reshape(nblk, 1, eb)
    didx = dst.reshape(nblk, 1, eb)

    bn = 1000 if n % 1000 == 0 else n
    ck = 1000 if n % 1000 == 0 else n

    a1s, a1d = _amat(a_src1, heads, hp), _amat(a_dst1, heads, hp)
    a2s, a2d = _amat(a_src2, heads, hp), _amat(a_dst2, heads, hp)
    a3s, a3d = _amat(a_src3, 1, hp), _amat(a_dst3, 1, hp)

    ha1 = _run_dense_first(x, W1, a1s, a1d, bn, d12, hp)
    num1, den1 = _run_edges(sidx, didx, ha1, n, d12, heads, hid, hp, eb, ck)

    ha2 = _run_dense_next(num1, den1, b1.reshape(1, -1), W2, a2s, a2d,
                          bn, heads, hid, d12, hp)
    num2, den2 = _run_edges(sidx, didx, ha2, n, d12, heads, hid, hp, eb, ck)

    ha3 = _run_dense_next(num2, den2, b2.reshape(1, -1), W3, a3s, a3d,
                          bn, heads, hid, out_d, hp)
    num3, den3 = _run_edges(sidx, didx, ha3, n, out_d, 1, out_d, hp, eb, ck)

    return _run_final(num3, den3, b3.reshape(1, -1), bn)


# speedup vs baseline: 4.6038x; 1.0866x over previous
import functools
import math

import jax
import jax.numpy as jnp
from jax.experimental import pallas as pl


def _amat(a, h_true, hp):
    h, c = a.shape[1], a.shape[2]
    eye = jnp.eye(h, dtype=a.dtype)
    m = jnp.einsum('hc,hg->hcg', a[0], eye).reshape(h * c, h)
    if h < hp:
        m = jnp.tile(m, (1, hp // h))
    return m


def _mm_first_kernel(x_ref, w_ref, asrc_ref, adst_ref, out_ref, *, d, hp):
    h = jnp.dot(x_ref[...], w_ref[...], preferred_element_type=jnp.float32)
    out_ref[:, :d] = h
    out_ref[:, d:d + hp] = jnp.dot(h, asrc_ref[...],
                                   preferred_element_type=jnp.float32)
    out_ref[:, d + hp:] = jnp.dot(h, adst_ref[...],
                                  preferred_element_type=jnp.float32)


def _mm_next_kernel(num_ref, den_ref, b_ref, w_ref, asrc_ref, adst_ref,
                    out_ref, *, hin, cin, d, hp):
    b = b_ref[...]
    parts = [
        jnp.maximum(
            num_ref[:, h0 * cin:(h0 + 1) * cin]
            / (den_ref[:, h0:h0 + 1] + 1e-16)
            + b[:, h0 * cin:(h0 + 1) * cin], 0.0)
        for h0 in range(hin)
    ]
    xin = jnp.concatenate(parts, axis=1)
    h = jnp.dot(xin, w_ref[...], preferred_element_type=jnp.float32)
    out_ref[:, :d] = h
    out_ref[:, d:d + hp] = jnp.dot(h, asrc_ref[...],
                                   preferred_element_type=jnp.float32)
    out_ref[:, d + hp:] = jnp.dot(h, adst_ref[...],
                                  preferred_element_type=jnp.float32)


def _edge_kernel(sidx_ref, didx_ref, ha_ref, num_ref, den_ref,
                 *, n, d, ht, c, hp, eb, ck):
    i = pl.program_id(0)

    @pl.when(i == 0)
    def _():
        num_ref[...] = jnp.zeros_like(num_ref)
        den_ref[...] = jnp.zeros_like(den_ref)

    s_b = sidx_ref[0]
    d_b = didx_ref[0]
    ew = eb + 8
    w0 = jnp.maximum(jnp.minimum((jnp.min(d_b) // 8) * 8, n - ew), 0)
    w0 = pl.multiple_of(w0, 8)

    wio = jax.lax.broadcasted_iota(jnp.int32, (ew, eb), 0) + w0
    ohd_t = (jnp.broadcast_to(d_b, (ew, eb)) == wio).astype(jnp.bfloat16)

    ad_w = ha_ref[pl.ds(w0, ew), d + hp:]
    ad_g = jax.lax.dot_general(ohd_t, ad_w.astype(jnp.bfloat16),
                               (((0,), (0,)), ((), ())),
                               preferred_element_type=jnp.float32)

    def body(k, acc):
        cio = jax.lax.broadcasted_iota(jnp.int32, (ck, eb), 0) + k * ck
        ohs_t = (jnp.broadcast_to(s_b, (ck, eb)) == cio).astype(jnp.bfloat16)
        blk = ha_ref[pl.ds(pl.multiple_of(k * ck, 8), ck), :d + hp]
        return acc + jax.lax.dot_general(
            ohs_t, blk.astype(jnp.bfloat16), (((0,), (0,)), ((), ())),
            preferred_element_type=jnp.float32)

    g = jax.lax.fori_loop(0, n // ck, body,
                          jnp.zeros((eb, d + hp), jnp.float32))
    h_g = g[:, :d]
    as_g = g[:, d:]

    e = as_g + ad_g
    e = jnp.where(e >= 0, e, 0.2 * e)
    ex = jnp.exp(e)

    parts = [h_g[:, h0 * c:(h0 + 1) * c] * ex[:, h0:h0 + 1]
             for h0 in range(ht)]
    weighted = parts[0] if ht == 1 else jnp.concatenate(parts, axis=1)

    num_ref[pl.ds(w0, ew), :] += jax.lax.dot_general(
        ohd_t, weighted.astype(jnp.bfloat16), (((1,), (0,)), ((), ())),
        preferred_element_type=jnp.float32)
    den_ref[pl.ds(w0, ew), :] += jax.lax.dot_general(
        ohd_t, ex.astype(jnp.bfloat16), (((1,), (0,)), ((), ())),
        preferred_element_type=jnp.float32)


def _final_kernel(num_ref, den_ref, b_ref, out_ref):
    l = num_ref[...] / (den_ref[:, 0:1] + 1e-16) + b_ref[...]
    m = jnp.max(l, axis=1, keepdims=True)
    lse = jnp.log(jnp.sum(jnp.exp(l - m), axis=1, keepdims=True)) + m
    out_ref[...] = l - lse


def _run_dense_first(x, w, asrc, adst, bn, d, hp):
    n, fin = x.shape
    grid = n // bn
    return pl.pallas_call(
        functools.partial(_mm_first_kernel, d=d, hp=hp),
        grid=(grid,),
        in_specs=[
            pl.BlockSpec((bn, fin), lambda i: (i, 0)),
            pl.BlockSpec((fin, d), lambda i: (0, 0)),
            pl.BlockSpec((d, hp), lambda i: (0, 0)),
            pl.BlockSpec((d, hp), lambda i: (0, 0)),
        ],
        out_specs=pl.BlockSpec((bn, d + 2 * hp), lambda i: (i, 0)),
        out_shape=jax.ShapeDtypeStruct((n, d + 2 * hp), jnp.float32),
    )(x, w, asrc, adst)


def _run_dense_next(num, den, b, w, asrc, adst, bn, hin, cin, d, hp):
    n, din = num.shape
    grid = n // bn
    return pl.pallas_call(
        functools.partial(_mm_next_kernel, hin=hin, cin=cin, d=d, hp=hp),
        grid=(grid,),
        in_specs=[
            pl.BlockSpec((bn, din), lambda i: (i, 0)),
            pl.BlockSpec((bn, hp), lambda i: (i, 0)),
            pl.BlockSpec((1, din), lambda i: (0, 0)),
            pl.BlockSpec((din, d), lambda i: (0, 0)),
            pl.BlockSpec((d, hp), lambda i: (0, 0)),
            pl.BlockSpec((d, hp), lambda i: (0, 0)),
        ],
        out_specs=pl.BlockSpec((bn, d + 2 * hp), lambda i: (i, 0)),
        out_shape=jax.ShapeDtypeStruct((n, d + 2 * hp), jnp.float32),
    )(num, den, b, w, asrc, adst)


def _run_edges(sidx, didx, ha, n, d, ht, c, hp, eb, ck):
    nblk = sidx.shape[0]
    return pl.pallas_call(
        functools.partial(_edge_kernel, n=n, d=d, ht=ht, c=c, hp=hp,
                          eb=eb, ck=ck),
        grid=(nblk,),
        in_specs=[
            pl.BlockSpec((1, 1, eb), lambda i: (i, 0, 0)),
            pl.BlockSpec((1, 1, eb), lambda i: (i, 0, 0)),
            pl.BlockSpec((n, d + 2 * hp), lambda i: (0, 0)),
        ],
        out_specs=[
            pl.BlockSpec((n, d), lambda i: (0, 0)),
            pl.BlockSpec((n, hp), lambda i: (0, 0)),
        ],
        out_shape=[
            jax.ShapeDtypeStruct((n, d), jnp.float32),
            jax.ShapeDtypeStruct((n, hp), jnp.float32),
        ],
    )(sidx, didx, ha)


def _run_final(num, den, b, bn):
    n, d = num.shape
    hp = den.shape[1]
    grid = n // bn
    return pl.pallas_call(
        _final_kernel,
        grid=(grid,),
        in_specs=[
            pl.BlockSpec((bn, d), lambda i: (i, 0)),
            pl.BlockSpec((bn, hp), lambda i: (i, 0)),
            pl.BlockSpec((1, d), lambda i: (0, 0)),
        ],
        out_specs=pl.BlockSpec((bn, d), lambda i: (i, 0)),
        out_shape=jax.ShapeDtypeStruct((n, d), jnp.float32),
    )(num, den, b)


def kernel(x, edge_index, W1, a_src1, a_dst1, b1, W2, a_src2, a_dst2, b2,
           W3, a_src3, a_dst3, b3):
    n = x.shape[0]
    e = edge_index.shape[1]
    hp = 8
    heads = a_src1.shape[1]
    hid = a_src1.shape[2]
    out_d = a_src3.shape[2]
    d12 = heads * hid

    loop = jnp.arange(n, dtype=edge_index.dtype)
    src = jnp.concatenate([edge_index[0], loop])
    dst = jnp.concatenate([edge_index[1], loop])
    order = jnp.argsort(dst)
    src = src[order]
    dst = dst[order]

    eb = 1024 if n >= 2048 else min(n, 512)
    total = e + n
    nblk = math.ceil(total / eb)
    pad = nblk * eb - total
    src = jnp.concatenate([src, jnp.zeros((pad,), src.dtype)])
    dst = jnp.concatenate([dst, jnp.full((pad,), n, dst.dtype)])
    sidx = src.reshape(nblk, 1, eb)
    didx = dst.reshape(nblk, 1, eb)

    bn = 1000 if n % 1000 == 0 else n
    ck = 1000 if n % 1000 == 0 else n

    a1s, a1d = _amat(a_src1, heads, hp), _amat(a_dst1, heads, hp)
    a2s, a2d = _amat(a_src2, heads, hp), _amat(a_dst2, heads, hp)
    a3s, a3d = _amat(a_src3, 1, hp), _amat(a_dst3, 1, hp)

    ha1 = _run_dense_first(x, W1, a1s, a1d, bn, d12, hp)
    num1, den1 = _run_edges(sidx, didx, ha1, n, d12, heads, hid, hp, eb, ck)

    ha2 = _run_dense_next(num1, den1, b1.reshape(1, -1), W2, a2s, a2d,
                          bn, heads, hid, d12, hp)
    num2, den2 = _run_edges(sidx, didx, ha2, n, d12, heads, hid, hp, eb, ck)

    ha3 = _run_dense_next(num2, den2, b2.reshape(1, -1), W3, a3s, a3d,
                          bn, heads, hid, out_d, hp)
    num3, den3 = _run_edges(sidx, didx, ha3, n, out_d, 1, out_d, hp, eb, ck)

    return _run_final(num3, den3, b3.reshape(1, -1), bn)
